# baseline (speedup 1.0000x reference)
import functools

import jax
import jax.numpy as jnp
from jax import lax
from jax.experimental import pallas as pl
from jax.experimental.pallas import tpu as pltpu
from jax.experimental.pallas import tpu_sc as plsc

_N = 10000
_E = 320000
_F = 128
_HID = 64
_C = 40
_K = 10
_ALPHA = 0.1

_NT = 16
_CHUNK = 128
_GSZ = 16
_NGRP = 10
_NCH = _GSZ * _NGRP
_EPT = _NCH * _CHUNK
_EPAD = _NT * _EPT
_NPAD = 10240
_CPAD = 48
_ROWS_PT = _NPAD // _NT
_UCH = 128
_NB = 4
_UPC = _ROWS_PT // _UCH

_mesh = plsc.VectorSubcoreMesh(
    core_axis_name="c", subcore_axis_name="s", num_cores=1
)
_sc_params = pltpu.CompilerParams(use_tc_tiling_on_sc=False)


@functools.partial(
    pl.kernel,
    out_type=jax.ShapeDtypeStruct((_NPAD,), jnp.float32),
    mesh=_mesh,
    compiler_params=_sc_params,
    scratch_types=[
        pltpu.VMEM_SHARED((_NPAD,), jnp.float32),
        pltpu.VMEM((_GSZ, _CHUNK), jnp.int32),
        pltpu.VMEM((_ROWS_PT,), jnp.float32),
    ],
)
def _deg_kernel(dst_hbm, deg_hbm, deg_sh, didx, ones):
    tid = lax.axis_index("s")
    row0 = tid * _ROWS_PT

    def fill(i, c):
        ones[pl.ds(i * 16, 16)] = jnp.full((16,), 1.0, jnp.float32)
        return c

    lax.fori_loop(0, _ROWS_PT // 16, fill, 0)
    pltpu.sync_copy(ones, deg_sh.at[pl.ds(row0, _ROWS_PT)])
    plsc.subcore_barrier()

    def group(g, c):
        pltpu.sync_copy(dst_hbm.at[tid, pl.ds(g * _GSZ, _GSZ)], didx)

        def chunk(j, cc):
            pltpu.sync_copy(
                ones.at[pl.ds(0, _CHUNK)], deg_sh.at[didx.at[j]], add=True)
            return cc

        lax.fori_loop(0, _GSZ, chunk, 0)
        return c

    lax.fori_loop(0, _NGRP, group, 0)
    plsc.subcore_barrier()
    pltpu.sync_copy(deg_sh.at[pl.ds(row0, _ROWS_PT)], deg_hbm.at[pl.ds(row0, _ROWS_PT)])


def _prep_body(x_ref, w1_ref, b1_ref, w2_ref, b2_ref, deg_ref,
               a_ref, b_ref, a2_ref, b2_ref_o, q0_ref):
    h = jnp.maximum(
        jax.lax.dot_general(
            x_ref[...], w1_ref[...], (((1,), (0,)), ((), ())),
            preferred_element_type=jnp.float32,
        ) + b1_ref[...], 0.0)
    h = jax.lax.dot_general(
        h, w2_ref[...], (((1,), (0,)), ((), ())),
        preferred_element_type=jnp.float32,
    ) + b2_ref[...]
    blk = h.shape[0]
    hp = jnp.concatenate([h, jnp.zeros((blk, _CPAD - _C), jnp.float32)], axis=1)
    dinv = jax.lax.rsqrt(deg_ref[...])
    a_ref[...] = jnp.broadcast_to((1.0 - _ALPHA) * dinv * dinv, (blk, _CPAD))
    a2_ref[...] = jnp.broadcast_to((1.0 - _ALPHA) * dinv, (blk, _CPAD))
    b_ref[...] = _ALPHA * dinv * hp
    b2_ref_o[...] = _ALPHA * hp
    q0_ref[...] = dinv * hp


_PREP_BLK = 2048


def _prep_call(x_p, W1, b1, W2, b2, deg2):
    grid = (_NPAD // _PREP_BLK,)
    shp = jax.ShapeDtypeStruct((_NPAD, _CPAD), jnp.float32)
    return pl.pallas_call(
        _prep_body,
        grid=grid,
        in_specs=[
            pl.BlockSpec((_PREP_BLK, _F), lambda i: (i, 0)),
            pl.BlockSpec((_F, _HID), lambda i: (0, 0)),
            pl.BlockSpec((1, _HID), lambda i: (0, 0)),
            pl.BlockSpec((_HID, _C), lambda i: (0, 0)),
            pl.BlockSpec((1, _C), lambda i: (0, 0)),
            pl.BlockSpec((_PREP_BLK, 1), lambda i: (i, 0)),
        ],
        out_specs=[pl.BlockSpec((_PREP_BLK, _CPAD), lambda i: (i, 0))] * 5,
        out_shape=[shp] * 5,
    )(x_p, W1, b1, W2, b2, deg2)


@functools.partial(
    pl.kernel,
    out_type=(jax.ShapeDtypeStruct((_NPAD, _CPAD), jnp.float32),
              jax.ShapeDtypeStruct((_NPAD, _CPAD), jnp.float32)),
    mesh=_mesh,
    compiler_params=_sc_params,
    scratch_types=[
        pltpu.VMEM_SHARED((_NPAD, _CPAD), jnp.float32),
        pltpu.VMEM((_NCH, _CHUNK), jnp.int32),
        pltpu.VMEM((_NB, _CHUNK), jnp.int32),
        pltpu.VMEM((_NB, _CHUNK), jnp.int32),
        pltpu.VMEM((_NB, _CHUNK, _CPAD), jnp.float32),
        pltpu.SemaphoreType.DMA((_NB,)),
        pltpu.SemaphoreType.DMA((_NB,)),
        pltpu.VMEM((2, _UCH, _CPAD), jnp.float32),
        pltpu.VMEM((2, _UCH, _CPAD), jnp.float32),
        pltpu.VMEM((2, _UCH, _CPAD), jnp.float32),
        pltpu.VMEM((2, _UCH, _CPAD), jnp.float32),
        pltpu.SemaphoreType.DMA((2,)),
        pltpu.SemaphoreType.DMA((2,)),
        pltpu.SemaphoreType.DMA((2,)),
        pltpu.SemaphoreType.DMA((2,)),
        pltpu.SemaphoreType.DMA((2,)),
    ],
)
def _prop_kernel(pidx_hbm, a_hbm, b_hbm, a2_hbm, b2_hbm, q0_hbm,
                 out_hbm, qw_hbm, s_sh, pidx, srcu, dstu, gbuf, gsem, ssem,
                 uS, uA, uB, uQ, lsemS, lsemA, lsemB, qsem, psem):
    tid = lax.axis_index("s")
    row0 = tid * _ROWS_PT

    pltpu.sync_copy(pidx_hbm.at[tid], pidx)

    def initc(c, carry):
        r = row0 + c * _UCH
        pltpu.sync_copy(q0_hbm.at[pl.ds(r, _UCH)], uQ.at[0])
        pltpu.sync_copy(uQ.at[0], qw_hbm.at[pl.ds(r, _UCH)])
        pltpu.sync_copy(uQ.at[0], s_sh.at[pl.ds(r, _UCH)])
        return carry

    lax.fori_loop(0, _UPC, initc, 0)
    plsc.subcore_barrier()

    def unpack(j, brow):
        for v in range(_CHUNK // 16):
            sl = pl.ds(v * 16, 16)
            p = pidx[j, sl]
            srcu[brow, sl] = lax.shift_right_logical(p, 14)
            dstu[brow, sl] = lax.bitwise_and(p, 16383)

    def edge_pass():
        unpack(0, 0)
        pltpu.async_copy(qw_hbm.at[srcu.at[0]], gbuf.at[0], gsem.at[0])

        def edge(j, carry):
            b = lax.rem(j, _NB)
            nb = lax.rem(j + 1, _NB)
            pltpu.make_async_copy(
                qw_hbm.at[srcu.at[b]], gbuf.at[b], gsem.at[b]).wait()
            pltpu.async_copy(gbuf.at[b], s_sh.at[dstu.at[b]], ssem.at[b],
                             add=True)

            @pl.when(j < _NCH - 1)
            def _():
                @pl.when(j >= _NB - 1)
                def _():
                    pltpu.make_async_copy(
                        gbuf.at[nb], s_sh.at[dstu.at[0]], ssem.at[nb]).wait()

                unpack(j + 1, nb)
                pltpu.async_copy(
                    qw_hbm.at[srcu.at[nb]], gbuf.at[nb], gsem.at[nb])

            return carry

        lax.fori_loop(0, _NCH, edge, 0)
        for jj in range(_NCH - _NB, _NCH):
            bb = jj % _NB
            pltpu.make_async_copy(
                gbuf.at[bb], s_sh.at[dstu.at[0]], ssem.at[bb]).wait()
        plsc.subcore_barrier()

    def update(a_src, b_src, dst, prime):
        def fire_loads(c, pb):
            r = row0 + c * _UCH
            pltpu.async_copy(s_sh.at[pl.ds(r, _UCH)], uS.at[pb], lsemS.at[pb])
            pltpu.async_copy(a_src.at[pl.ds(r, _UCH)], uA.at[pb], lsemA.at[pb])
            pltpu.async_copy(b_src.at[pl.ds(r, _UCH)], uB.at[pb], lsemB.at[pb])

        def wait_stores(pb):
            pltpu.make_async_copy(
                uQ.at[pb], dst.at[pl.ds(row0, _UCH)], qsem.at[pb]).wait()
            if prime:
                pltpu.make_async_copy(
                    uQ.at[pb], s_sh.at[pl.ds(row0, _UCH)], psem.at[pb]).wait()

        fire_loads(0, 0)

        def upd(c, carry):
            pb = lax.rem(c, 2)
            npb = 1 - pb
            r = row0 + c * _UCH

            @pl.when(c < _UPC - 1)
            def _():
                fire_loads(c + 1, npb)

            pltpu.make_async_copy(
                s_sh.at[pl.ds(row0, _UCH)], uS.at[pb], lsemS.at[pb]).wait()
            pltpu.make_async_copy(
                a_src.at[pl.ds(row0, _UCH)], uA.at[pb], lsemA.at[pb]).wait()
            pltpu.make_async_copy(
                b_src.at[pl.ds(row0, _UCH)], uB.at[pb], lsemB.at[pb]).wait()

            @pl.when(c >= 2)
            def _():
                wait_stores(pb)

            def rows(v, cc):
                for c3 in range(_CPAD // 16):
                    sl = pl.ds(c3 * 16, 16)
                    uQ[pb, v, sl] = uA[pb, v, sl] * uS[pb, v, sl] + uB[pb, v, sl]
                return cc

            lax.fori_loop(0, _UCH, rows, 0)
            pltpu.async_copy(uQ.at[pb], dst.at[pl.ds(r, _UCH)], qsem.at[pb])
            if prime:
                pltpu.async_copy(uQ.at[pb], s_sh.at[pl.ds(r, _UCH)],
                                 psem.at[pb])
            return carry

        lax.fori_loop(0, _UPC, upd, 0)
        for cc in (_UPC - 2, _UPC - 1):
            wait_stores(cc % 2)
        plsc.subcore_barrier()

    def step(k, carry):
        edge_pass()
        update(a_hbm, b_hbm, qw_hbm, prime=True)
        return carry

    lax.fori_loop(0, _K - 1, step, 0)
    edge_pass()
    update(a2_hbm, b2_hbm, out_hbm, prime=False)


@jax.jit
def kernel(x, edge_index, W1, b1, W2, b2):
    src = edge_index[0]
    dst = edge_index[1]
    npad_ids = (jnp.arange(_EPAD - _E, dtype=jnp.int32) % (_NPAD - _N)) + _N
    src_p = jnp.concatenate([src, npad_ids])
    dst_p = jnp.concatenate([dst, npad_ids])
    dst_r = dst_p.reshape(_NT, _NCH, _CHUNK)
    pidx_r = (src_p * 16384 + dst_p).reshape(_NT, _NCH, _CHUNK)

    deg = _deg_kernel(dst_r)

    x_p = jnp.concatenate(
        [x, jnp.zeros((_NPAD - _N, _F), jnp.float32)], axis=0)
    A, B, A2, B2, Q0 = _prep_call(
        x_p, W1, b1.reshape(1, _HID), W2, b2.reshape(1, _C),
        deg.reshape(_NPAD, 1))

    out, _ = _prop_kernel(pidx_r, A, B, A2, B2, Q0)
    return out[:_N, :_C]


# speedup vs baseline: 1.8573x; 1.8573x over previous
import functools

import jax
import jax.numpy as jnp
from jax import lax
from jax.experimental import pallas as pl
from jax.experimental.pallas import tpu as pltpu
from jax.experimental.pallas import tpu_sc as plsc

_N = 10000
_E = 320000
_F = 128
_HID = 64
_C = 40
_K = 10
_ALPHA = 0.1

_NT = 16
_CHUNK = 128
_GSZ = 16
_NGRP = 10
_NCH = _GSZ * _NGRP
_EPT = _NCH * _CHUNK
_EPAD = _NT * _EPT
_NPAD = 10240
_CPAD = 48
_CP2 = 64
_CH = 32
_ROWS_PT = _NPAD // _NT
_UCH = 128
_NB = 4
_UPC = _ROWS_PT // _UCH
_NC = 2

_mesh = plsc.VectorSubcoreMesh(
    core_axis_name="c", subcore_axis_name="s", num_cores=1
)
_mesh2 = plsc.VectorSubcoreMesh(
    core_axis_name="c", subcore_axis_name="s", num_cores=2
)
_sc_params = pltpu.CompilerParams(use_tc_tiling_on_sc=False)


@functools.partial(
    pl.kernel,
    out_type=jax.ShapeDtypeStruct((_NPAD,), jnp.float32),
    mesh=_mesh,
    compiler_params=_sc_params,
    scratch_types=[
        pltpu.VMEM_SHARED((_NPAD,), jnp.float32),
        pltpu.VMEM((_GSZ, _CHUNK), jnp.int32),
        pltpu.VMEM((_ROWS_PT,), jnp.float32),
    ],
)
def _deg_kernel(dst_hbm, deg_hbm, deg_sh, didx, ones):
    tid = lax.axis_index("s")
    row0 = tid * _ROWS_PT

    def fill(i, c):
        ones[pl.ds(i * 16, 16)] = jnp.full((16,), 1.0, jnp.float32)
        return c

    lax.fori_loop(0, _ROWS_PT // 16, fill, 0)
    pltpu.sync_copy(ones, deg_sh.at[pl.ds(row0, _ROWS_PT)])
    plsc.subcore_barrier()

    def group(g, c):
        pltpu.sync_copy(dst_hbm.at[tid, pl.ds(g * _GSZ, _GSZ)], didx)

        def chunk(j, cc):
            pltpu.sync_copy(
                ones.at[pl.ds(0, _CHUNK)], deg_sh.at[didx.at[j]], add=True)
            return cc

        lax.fori_loop(0, _GSZ, chunk, 0)
        return c

    lax.fori_loop(0, _NGRP, group, 0)
    plsc.subcore_barrier()
    pltpu.sync_copy(deg_sh.at[pl.ds(row0, _ROWS_PT)], deg_hbm.at[pl.ds(row0, _ROWS_PT)])


def _prep_body(x_ref, w1_ref, b1_ref, w2_ref, b2_ref, deg_ref,
               a_ref, b_ref, a2_ref, b2_ref_o, q0_ref):
    h = jnp.maximum(
        jax.lax.dot_general(
            x_ref[...], w1_ref[...], (((1,), (0,)), ((), ())),
            preferred_element_type=jnp.float32,
        ) + b1_ref[...], 0.0)
    h = jax.lax.dot_general(
        h, w2_ref[...], (((1,), (0,)), ((), ())),
        preferred_element_type=jnp.float32,
    ) + b2_ref[...]
    blk = h.shape[0]
    hp = jnp.concatenate([h, jnp.zeros((blk, _CP2 - _C), jnp.float32)], axis=1)
    dinv = jax.lax.rsqrt(deg_ref[...])
    a_ref[...] = jnp.broadcast_to((1.0 - _ALPHA) * dinv * dinv, (blk, _CP2))
    a2_ref[...] = jnp.broadcast_to((1.0 - _ALPHA) * dinv, (blk, _CP2))
    b_ref[...] = _ALPHA * dinv * hp
    b2_ref_o[...] = _ALPHA * hp
    q0_ref[...] = dinv * hp


_PREP_BLK = 2048


def _prep_call(x_p, W1, b1, W2, b2, deg2):
    grid = (_NPAD // _PREP_BLK,)
    shp = jax.ShapeDtypeStruct((_NPAD, _CP2), jnp.float32)
    return pl.pallas_call(
        _prep_body,
        grid=grid,
        in_specs=[
            pl.BlockSpec((_PREP_BLK, _F), lambda i: (i, 0)),
            pl.BlockSpec((_F, _HID), lambda i: (0, 0)),
            pl.BlockSpec((1, _HID), lambda i: (0, 0)),
            pl.BlockSpec((_HID, _C), lambda i: (0, 0)),
            pl.BlockSpec((1, _C), lambda i: (0, 0)),
            pl.BlockSpec((_PREP_BLK, 1), lambda i: (i, 0)),
        ],
        out_specs=[pl.BlockSpec((_PREP_BLK, _CP2), lambda i: (i, 0))] * 5,
        out_shape=[shp] * 5,
    )(x_p, W1, b1, W2, b2, deg2)


@functools.partial(
    pl.kernel,
    out_type=jax.ShapeDtypeStruct((_NC, _NPAD, _CH), jnp.float32),
    mesh=_mesh2,
    compiler_params=_sc_params,
    scratch_types=[
        pltpu.VMEM_SHARED((_NPAD, _CH), jnp.float32),
        pltpu.VMEM_SHARED((_NPAD, _CH), jnp.float32),
        pltpu.VMEM((_NCH, _CHUNK), jnp.int32),
        pltpu.VMEM((_NB, _CHUNK), jnp.int32),
        pltpu.VMEM((_NB, _CHUNK), jnp.int32),
        pltpu.VMEM((_NB, _CHUNK, _CH), jnp.float32),
        pltpu.SemaphoreType.DMA((_NB,)),
        pltpu.SemaphoreType.DMA((_NB,)),
        pltpu.VMEM((2, _UCH, _CH), jnp.float32),
        pltpu.VMEM((2, _UCH, _CH), jnp.float32),
        pltpu.VMEM((2, _UCH, _CH), jnp.float32),
        pltpu.VMEM((2, _UCH, _CH), jnp.float32),
        pltpu.SemaphoreType.DMA((2,)),
        pltpu.SemaphoreType.DMA((2,)),
        pltpu.SemaphoreType.DMA((2,)),
        pltpu.SemaphoreType.DMA((2,)),
        pltpu.SemaphoreType.DMA((2,)),
    ],
)
def _prop_kernel(pidx_hbm, a_hbm, b_hbm, a2_hbm, b2_hbm, q0_hbm,
                 out_hbm, q_sh, s_sh, pidx, srcu, dstu, gbuf, gsem, ssem,
                 uS, uA, uB, uQ, lsemS, lsemA, lsemB, qsem, psem):
    cid = lax.axis_index("c")
    tid = lax.axis_index("s")
    row0 = tid * _ROWS_PT

    pltpu.sync_copy(pidx_hbm.at[tid], pidx)

    def initc(c, carry):
        r = row0 + c * _UCH
        pltpu.sync_copy(q0_hbm.at[cid, pl.ds(r, _UCH)], uQ.at[0])
        pltpu.sync_copy(uQ.at[0], q_sh.at[pl.ds(r, _UCH)])
        pltpu.sync_copy(uQ.at[0], s_sh.at[pl.ds(r, _UCH)])
        return carry

    lax.fori_loop(0, _UPC, initc, 0)
    plsc.subcore_barrier()

    def unpack(j, brow):
        for v in range(_CHUNK // 16):
            sl = pl.ds(v * 16, 16)
            p = pidx[j, sl]
            srcu[brow, sl] = lax.shift_right_logical(p, 14)
            dstu[brow, sl] = lax.bitwise_and(p, 16383)

    def edge_pass():
        unpack(0, 0)
        pltpu.async_copy(q_sh.at[srcu.at[0]], gbuf.at[0], gsem.at[0])

        def edge(j, carry):
            b = lax.rem(j, _NB)
            nb = lax.rem(j + 1, _NB)
            pltpu.make_async_copy(
                q_sh.at[srcu.at[b]], gbuf.at[b], gsem.at[b]).wait()
            pltpu.async_copy(gbuf.at[b], s_sh.at[dstu.at[b]], ssem.at[b],
                             add=True)

            @pl.when(j < _NCH - 1)
            def _():
                @pl.when(j >= _NB - 1)
                def _():
                    pltpu.make_async_copy(
                        gbuf.at[nb], s_sh.at[dstu.at[0]], ssem.at[nb]).wait()

                unpack(j + 1, nb)
                pltpu.async_copy(
                    q_sh.at[srcu.at[nb]], gbuf.at[nb], gsem.at[nb])

            return carry

        lax.fori_loop(0, _NCH, edge, 0)
        for jj in range(_NCH - _NB, _NCH):
            bb = jj % _NB
            pltpu.make_async_copy(
                gbuf.at[bb], s_sh.at[dstu.at[0]], ssem.at[bb]).wait()
        plsc.subcore_barrier()

    def update(a_src, b_src, to_hbm):
        def fire_loads(c, pb):
            r = row0 + c * _UCH
            pltpu.async_copy(s_sh.at[pl.ds(r, _UCH)], uS.at[pb], lsemS.at[pb])
            pltpu.async_copy(a_src.at[cid, pl.ds(r, _UCH)], uA.at[pb],
                             lsemA.at[pb])
            pltpu.async_copy(b_src.at[cid, pl.ds(r, _UCH)], uB.at[pb],
                             lsemB.at[pb])

        def wait_stores(pb):
            if to_hbm:
                pltpu.make_async_copy(
                    uQ.at[pb], out_hbm.at[cid, pl.ds(row0, _UCH)],
                    qsem.at[pb]).wait()
            else:
                pltpu.make_async_copy(
                    uQ.at[pb], q_sh.at[pl.ds(row0, _UCH)], qsem.at[pb]).wait()
                pltpu.make_async_copy(
                    uQ.at[pb], s_sh.at[pl.ds(row0, _UCH)], psem.at[pb]).wait()

        fire_loads(0, 0)

        def upd(c, carry):
            pb = lax.rem(c, 2)
            npb = 1 - pb
            r = row0 + c * _UCH

            @pl.when(c < _UPC - 1)
            def _():
                fire_loads(c + 1, npb)

            pltpu.make_async_copy(
                s_sh.at[pl.ds(row0, _UCH)], uS.at[pb], lsemS.at[pb]).wait()
            pltpu.make_async_copy(
                a_src.at[cid, pl.ds(row0, _UCH)], uA.at[pb],
                lsemA.at[pb]).wait()
            pltpu.make_async_copy(
                b_src.at[cid, pl.ds(row0, _UCH)], uB.at[pb],
                lsemB.at[pb]).wait()

            @pl.when(c >= 2)
            def _():
                wait_stores(pb)

            def rows(v, cc):
                for c3 in range(_CH // 16):
                    sl = pl.ds(c3 * 16, 16)
                    uQ[pb, v, sl] = uA[pb, v, sl] * uS[pb, v, sl] + uB[pb, v, sl]
                return cc

            lax.fori_loop(0, _UCH, rows, 0)
            if to_hbm:
                pltpu.async_copy(uQ.at[pb], out_hbm.at[cid, pl.ds(r, _UCH)],
                                 qsem.at[pb])
            else:
                pltpu.async_copy(uQ.at[pb], q_sh.at[pl.ds(r, _UCH)],
                                 qsem.at[pb])
                pltpu.async_copy(uQ.at[pb], s_sh.at[pl.ds(r, _UCH)],
                                 psem.at[pb])
            return carry

        lax.fori_loop(0, _UPC, upd, 0)
        for cc in (_UPC - 2, _UPC - 1):
            wait_stores(cc % 2)
        plsc.subcore_barrier()

    def step(k, carry):
        edge_pass()
        update(a_hbm, b_hbm, to_hbm=False)
        return carry

    lax.fori_loop(0, _K - 1, step, 0)
    edge_pass()
    update(a2_hbm, b2_hbm, to_hbm=True)


@jax.jit
def kernel(x, edge_index, W1, b1, W2, b2):
    src = edge_index[0]
    dst = edge_index[1]
    npad_ids = (jnp.arange(_EPAD - _E, dtype=jnp.int32) % (_NPAD - _N)) + _N
    src_p = jnp.concatenate([src, npad_ids])
    dst_p = jnp.concatenate([dst, npad_ids])
    dst_r = dst_p.reshape(_NT, _NCH, _CHUNK)
    pidx_r = (src_p * 16384 + dst_p).reshape(_NT, _NCH, _CHUNK)

    deg = _deg_kernel(dst_r)

    x_p = jnp.concatenate(
        [x, jnp.zeros((_NPAD - _N, _F), jnp.float32)], axis=0)
    A, B, A2, B2, Q0 = _prep_call(
        x_p, W1, b1.reshape(1, _HID), W2, b2.reshape(1, _C),
        deg.reshape(_NPAD, 1))

    def halves(arr):
        return jnp.stack([arr[:, :_CH], arr[:, _CH:]])

    out2 = _prop_kernel(pidx_r, halves(A), halves(B), halves(A2),
                        halves(B2), halves(Q0))
    return jnp.concatenate([out2[0], out2[1]], axis=1)[:_N, :_C]


# speedup vs baseline: 1.9248x; 1.0363x over previous
import functools

import jax
import jax.numpy as jnp
from jax import lax
from jax.experimental import pallas as pl
from jax.experimental.pallas import tpu as pltpu
from jax.experimental.pallas import tpu_sc as plsc

_N = 10000
_E = 320000
_F = 128
_HID = 64
_C = 40
_K = 10
_ALPHA = 0.1

_NT = 16
_CHUNK = 128
_GSZ = 16
_NGRP = 10
_NCH = _GSZ * _NGRP
_EPT = _NCH * _CHUNK
_EPAD = _NT * _EPT
_NPAD = 10240
_CPAD = 48
_CP2 = 64
_CH = 32
_ROWS_PT = _NPAD // _NT
_UCH = 128
_NB = 4
_UPC = _ROWS_PT // _UCH
_NC = 2

_mesh = plsc.VectorSubcoreMesh(
    core_axis_name="c", subcore_axis_name="s", num_cores=1
)
_mesh2 = plsc.VectorSubcoreMesh(
    core_axis_name="c", subcore_axis_name="s", num_cores=2
)
_sc_params = pltpu.CompilerParams(use_tc_tiling_on_sc=False)


@functools.partial(
    pl.kernel,
    out_type=jax.ShapeDtypeStruct((_NPAD,), jnp.float32),
    mesh=_mesh,
    compiler_params=_sc_params,
    scratch_types=[
        pltpu.VMEM_SHARED((_NPAD,), jnp.float32),
        pltpu.VMEM((_GSZ, _CHUNK), jnp.int32),
        pltpu.VMEM((_ROWS_PT,), jnp.float32),
    ],
)
def _deg_kernel(dst_hbm, deg_hbm, deg_sh, didx, ones):
    tid = lax.axis_index("s")
    row0 = tid * _ROWS_PT

    def fill(i, c):
        ones[pl.ds(i * 16, 16)] = jnp.full((16,), 1.0, jnp.float32)
        return c

    lax.fori_loop(0, _ROWS_PT // 16, fill, 0)
    pltpu.sync_copy(ones, deg_sh.at[pl.ds(row0, _ROWS_PT)])
    plsc.subcore_barrier()

    def group(g, c):
        pltpu.sync_copy(dst_hbm.at[tid, pl.ds(g * _GSZ, _GSZ)], didx)

        def chunk(j, cc):
            pltpu.sync_copy(
                ones.at[pl.ds(0, _CHUNK)], deg_sh.at[didx.at[j]], add=True)
            return cc

        lax.fori_loop(0, _GSZ, chunk, 0)
        return c

    lax.fori_loop(0, _NGRP, group, 0)
    plsc.subcore_barrier()
    pltpu.sync_copy(deg_sh.at[pl.ds(row0, _ROWS_PT)], deg_hbm.at[pl.ds(row0, _ROWS_PT)])


def _prep_body(x_ref, w1_ref, b1_ref, w2_ref, b2_ref, deg_ref,
               a_ref, b_ref, a2_ref, b2_ref_o, q0_ref):
    h = jnp.maximum(
        jax.lax.dot_general(
            x_ref[...], w1_ref[...], (((1,), (0,)), ((), ())),
            preferred_element_type=jnp.float32,
        ) + b1_ref[...], 0.0)
    h = jax.lax.dot_general(
        h, w2_ref[...], (((1,), (0,)), ((), ())),
        preferred_element_type=jnp.float32,
    ) + b2_ref[...]
    blk = h.shape[0]
    hp = jnp.concatenate([h, jnp.zeros((blk, _CP2 - _C), jnp.float32)], axis=1)
    dinv = jax.lax.rsqrt(deg_ref[...])
    a_ref[...] = jnp.broadcast_to((1.0 - _ALPHA) * dinv * dinv, (blk, _CP2))
    a2_ref[...] = jnp.broadcast_to((1.0 - _ALPHA) * dinv, (blk, _CP2))
    b_ref[...] = _ALPHA * dinv * hp
    b2_ref_o[...] = _ALPHA * hp
    q0_ref[...] = dinv * hp


_PREP_BLK = 2048


def _prep_call(x_p, W1, b1, W2, b2, deg2):
    grid = (_NPAD // _PREP_BLK,)
    shp = jax.ShapeDtypeStruct((_NPAD, _CP2), jnp.float32)
    return pl.pallas_call(
        _prep_body,
        grid=grid,
        in_specs=[
            pl.BlockSpec((_PREP_BLK, _F), lambda i: (i, 0)),
            pl.BlockSpec((_F, _HID), lambda i: (0, 0)),
            pl.BlockSpec((1, _HID), lambda i: (0, 0)),
            pl.BlockSpec((_HID, _C), lambda i: (0, 0)),
            pl.BlockSpec((1, _C), lambda i: (0, 0)),
            pl.BlockSpec((_PREP_BLK, 1), lambda i: (i, 0)),
        ],
        out_specs=[pl.BlockSpec((_PREP_BLK, _CP2), lambda i: (i, 0))] * 5,
        out_shape=[shp] * 5,
    )(x_p, W1, b1, W2, b2, deg2)


@functools.partial(
    pl.kernel,
    out_type=jax.ShapeDtypeStruct((_NC, _NPAD, _CH), jnp.float32),
    mesh=_mesh2,
    compiler_params=_sc_params,
    scratch_types=[
        pltpu.VMEM_SHARED((_NPAD, _CH), jnp.float32),
        pltpu.VMEM_SHARED((_NPAD, _CH), jnp.float32),
        pltpu.VMEM((_NCH, _CHUNK), jnp.int32),
        pltpu.VMEM((_NB, 2 * _CHUNK), jnp.int32),
        pltpu.VMEM((_NB, 2 * _CHUNK), jnp.int32),
        pltpu.VMEM((_NB, 2 * _CHUNK, _CH), jnp.float32),
        pltpu.SemaphoreType.DMA((_NB,)),
        pltpu.SemaphoreType.DMA((_NB,)),
        pltpu.VMEM((2, _UCH, _CH), jnp.float32),
        pltpu.VMEM((2, _UCH, _CH), jnp.float32),
        pltpu.VMEM((2, _UCH, _CH), jnp.float32),
        pltpu.VMEM((2, _UCH, _CH), jnp.float32),
        pltpu.SemaphoreType.DMA((2,)),
        pltpu.SemaphoreType.DMA((2,)),
        pltpu.SemaphoreType.DMA((2,)),
        pltpu.SemaphoreType.DMA((2,)),
        pltpu.SemaphoreType.DMA((2,)),
    ],
)
def _prop_kernel(pidx_hbm, a_hbm, b_hbm, a2_hbm, b2_hbm, q0_hbm,
                 out_hbm, q_sh, s_sh, pidx, srcu, dstu, gbuf, gsem, ssem,
                 uS, uA, uB, uQ, lsemS, lsemA, lsemB, qsem, psem):
    cid = lax.axis_index("c")
    tid = lax.axis_index("s")
    row0 = tid * _ROWS_PT

    pltpu.sync_copy(pidx_hbm.at[tid], pidx)

    def initc(c, carry):
        r = row0 + c * _UCH
        pltpu.sync_copy(q0_hbm.at[cid, pl.ds(r, _UCH)], uQ.at[0])
        pltpu.sync_copy(uQ.at[0], q_sh.at[pl.ds(r, _UCH)])
        pltpu.sync_copy(uQ.at[0], s_sh.at[pl.ds(r, _UCH)])
        return carry

    lax.fori_loop(0, _UPC, initc, 0)
    plsc.subcore_barrier()

    def unpack(j, brow):
        for h in range(2):
            for v in range(_CHUNK // 16):
                sl = pl.ds(v * 16, 16)
                osl = pl.ds(h * _CHUNK + v * 16, 16)
                p = pidx[2 * j + h, sl]
                srcu[brow, osl] = lax.shift_right_logical(p, 14)
                dstu[brow, osl] = lax.bitwise_and(p, 16383)

    def edge_pass():
        unpack(0, 0)
        pltpu.async_copy(q_sh.at[srcu.at[0]], gbuf.at[0], gsem.at[0])

        def edge(j, carry):
            b = lax.rem(j, _NB)
            nb = lax.rem(j + 1, _NB)
            pltpu.make_async_copy(
                q_sh.at[srcu.at[b]], gbuf.at[b], gsem.at[b]).wait()
            pltpu.async_copy(gbuf.at[b], s_sh.at[dstu.at[b]], ssem.at[b],
                             add=True)

            @pl.when(j < _NCH // 2 - 1)
            def _():
                @pl.when(j >= _NB - 1)
                def _():
                    pltpu.make_async_copy(
                        gbuf.at[nb], s_sh.at[dstu.at[0]], ssem.at[nb]).wait()

                unpack(j + 1, nb)
                pltpu.async_copy(
                    q_sh.at[srcu.at[nb]], gbuf.at[nb], gsem.at[nb])

            return carry

        lax.fori_loop(0, _NCH // 2, edge, 0)
        for jj in range(_NCH // 2 - _NB, _NCH // 2):
            bb = jj % _NB
            pltpu.make_async_copy(
                gbuf.at[bb], s_sh.at[dstu.at[0]], ssem.at[bb]).wait()
        plsc.subcore_barrier()

    def update(a_src, b_src, to_hbm):
        def fire_loads(c, pb):
            r = row0 + c * _UCH
            pltpu.async_copy(s_sh.at[pl.ds(r, _UCH)], uS.at[pb], lsemS.at[pb])
            pltpu.async_copy(a_src.at[cid, pl.ds(r, _UCH)], uA.at[pb],
                             lsemA.at[pb])
            pltpu.async_copy(b_src.at[cid, pl.ds(r, _UCH)], uB.at[pb],
                             lsemB.at[pb])

        def wait_stores(pb):
            if to_hbm:
                pltpu.make_async_copy(
                    uQ.at[pb], out_hbm.at[cid, pl.ds(row0, _UCH)],
                    qsem.at[pb]).wait()
            else:
                pltpu.make_async_copy(
                    uQ.at[pb], q_sh.at[pl.ds(row0, _UCH)], qsem.at[pb]).wait()
                pltpu.make_async_copy(
                    uQ.at[pb], s_sh.at[pl.ds(row0, _UCH)], psem.at[pb]).wait()

        fire_loads(0, 0)

        def upd(c, carry):
            pb = lax.rem(c, 2)
            npb = 1 - pb
            r = row0 + c * _UCH

            @pl.when(c < _UPC - 1)
            def _():
                fire_loads(c + 1, npb)

            pltpu.make_async_copy(
                s_sh.at[pl.ds(row0, _UCH)], uS.at[pb], lsemS.at[pb]).wait()
            pltpu.make_async_copy(
                a_src.at[cid, pl.ds(row0, _UCH)], uA.at[pb],
                lsemA.at[pb]).wait()
            pltpu.make_async_copy(
                b_src.at[cid, pl.ds(row0, _UCH)], uB.at[pb],
                lsemB.at[pb]).wait()

            @pl.when(c >= 2)
            def _():
                wait_stores(pb)

            def rows(v, cc):
                for c3 in range(_CH // 16):
                    sl = pl.ds(c3 * 16, 16)
                    uQ[pb, v, sl] = uA[pb, v, sl] * uS[pb, v, sl] + uB[pb, v, sl]
                return cc

            lax.fori_loop(0, _UCH, rows, 0)
            if to_hbm:
                pltpu.async_copy(uQ.at[pb], out_hbm.at[cid, pl.ds(r, _UCH)],
                                 qsem.at[pb])
            else:
                pltpu.async_copy(uQ.at[pb], q_sh.at[pl.ds(r, _UCH)],
                                 qsem.at[pb])
                pltpu.async_copy(uQ.at[pb], s_sh.at[pl.ds(r, _UCH)],
                                 psem.at[pb])
            return carry

        lax.fori_loop(0, _UPC, upd, 0)
        for cc in (_UPC - 2, _UPC - 1):
            wait_stores(cc % 2)
        plsc.subcore_barrier()

    def step(k, carry):
        edge_pass()
        update(a_hbm, b_hbm, to_hbm=False)
        return carry

    lax.fori_loop(0, _K - 1, step, 0)
    edge_pass()
    update(a2_hbm, b2_hbm, to_hbm=True)


@jax.jit
def kernel(x, edge_index, W1, b1, W2, b2):
    src = edge_index[0]
    dst = edge_index[1]
    npad_ids = (jnp.arange(_EPAD - _E, dtype=jnp.int32) % (_NPAD - _N)) + _N
    src_p = jnp.concatenate([src, npad_ids])
    dst_p = jnp.concatenate([dst, npad_ids])
    dst_r = dst_p.reshape(_NT, _NCH, _CHUNK)
    pidx_r = (src_p * 16384 + dst_p).reshape(_NT, _NCH, _CHUNK)

    deg = _deg_kernel(dst_r)

    x_p = jnp.concatenate(
        [x, jnp.zeros((_NPAD - _N, _F), jnp.float32)], axis=0)
    A, B, A2, B2, Q0 = _prep_call(
        x_p, W1, b1.reshape(1, _HID), W2, b2.reshape(1, _C),
        deg.reshape(_NPAD, 1))

    def halves(arr):
        return jnp.stack([arr[:, :_CH], arr[:, _CH:]])

    out2 = _prop_kernel(pidx_r, halves(A), halves(B), halves(A2),
                        halves(B2), halves(Q0))
    return jnp.concatenate([out2[0], out2[1]], axis=1)[:_N, :_C]


# speedup vs baseline: 2.0341x; 1.0568x over previous
import functools

import jax
import jax.numpy as jnp
from jax import lax
from jax.experimental import pallas as pl
from jax.experimental.pallas import tpu as pltpu
from jax.experimental.pallas import tpu_sc as plsc

_N = 10000
_E = 320000
_F = 128
_HID = 64
_C = 40
_K = 10
_ALPHA = 0.1

_NT = 16
_CHUNK = 128
_GSZ = 16
_NGRP = 10
_NCH = _GSZ * _NGRP
_EPT = _NCH * _CHUNK
_EPAD = _NT * _EPT
_NPAD = 10240
_CPAD = 48
_CP2 = 64
_CH = 32
_ROWS_PT = _NPAD // _NT
_UCH = 128
_NB = 4
_UPC = _ROWS_PT // _UCH
_NC = 2

_mesh = plsc.VectorSubcoreMesh(
    core_axis_name="c", subcore_axis_name="s", num_cores=1
)
_mesh2 = plsc.VectorSubcoreMesh(
    core_axis_name="c", subcore_axis_name="s", num_cores=2
)
_sc_params = pltpu.CompilerParams(use_tc_tiling_on_sc=False)


@functools.partial(
    pl.kernel,
    out_type=jax.ShapeDtypeStruct((_NPAD,), jnp.float32),
    mesh=_mesh,
    compiler_params=_sc_params,
    scratch_types=[
        pltpu.VMEM_SHARED((_NPAD,), jnp.float32),
        pltpu.VMEM((_GSZ, _CHUNK), jnp.int32),
        pltpu.VMEM((_ROWS_PT,), jnp.float32),
    ],
)
def _deg_kernel(dst_hbm, deg_hbm, deg_sh, didx, ones):
    tid = lax.axis_index("s")
    row0 = tid * _ROWS_PT

    def fill(i, c):
        ones[pl.ds(i * 16, 16)] = jnp.full((16,), 1.0, jnp.float32)
        return c

    lax.fori_loop(0, _ROWS_PT // 16, fill, 0)
    pltpu.sync_copy(ones, deg_sh.at[pl.ds(row0, _ROWS_PT)])
    plsc.subcore_barrier()

    def group(g, c):
        pltpu.sync_copy(dst_hbm.at[tid, pl.ds(g * _GSZ, _GSZ)], didx)

        def chunk(j, cc):
            pltpu.sync_copy(
                ones.at[pl.ds(0, _CHUNK)], deg_sh.at[didx.at[j]], add=True)
            return cc

        lax.fori_loop(0, _GSZ, chunk, 0)
        return c

    lax.fori_loop(0, _NGRP, group, 0)
    plsc.subcore_barrier()
    pltpu.sync_copy(deg_sh.at[pl.ds(row0, _ROWS_PT)], deg_hbm.at[pl.ds(row0, _ROWS_PT)])


def _prep_body(x_ref, w1_ref, b1_ref, w2_ref, b2_ref, deg_ref,
               a_ref, b_ref, a2_ref, b2_ref_o, q0_ref):
    h = jnp.maximum(
        jax.lax.dot_general(
            x_ref[...], w1_ref[...], (((1,), (0,)), ((), ())),
            preferred_element_type=jnp.float32,
        ) + b1_ref[...], 0.0)
    h = jax.lax.dot_general(
        h, w2_ref[...], (((1,), (0,)), ((), ())),
        preferred_element_type=jnp.float32,
    ) + b2_ref[...]
    blk = h.shape[0]
    hp = jnp.concatenate([h, jnp.zeros((blk, _CP2 - _C), jnp.float32)], axis=1)
    dinv = jax.lax.rsqrt(deg_ref[...])

    def split(arr):
        return jnp.stack([arr[:, :_CH], arr[:, _CH:]], axis=0)

    a_ref[...] = split(jnp.broadcast_to((1.0 - _ALPHA) * dinv * dinv,
                                        (blk, _CP2)))
    a2_ref[...] = split(jnp.broadcast_to((1.0 - _ALPHA) * dinv, (blk, _CP2)))
    b_ref[...] = split(_ALPHA * dinv * hp)
    b2_ref_o[...] = split(_ALPHA * hp)
    q0_ref[...] = split(dinv * hp)


_PREP_BLK = 2048


def _prep_call(x_p, W1, b1, W2, b2, deg2):
    grid = (_NPAD // _PREP_BLK,)
    shp = jax.ShapeDtypeStruct((_NC, _NPAD, _CH), jnp.float32)
    return pl.pallas_call(
        _prep_body,
        grid=grid,
        in_specs=[
            pl.BlockSpec((_PREP_BLK, _F), lambda i: (i, 0)),
            pl.BlockSpec((_F, _HID), lambda i: (0, 0)),
            pl.BlockSpec((1, _HID), lambda i: (0, 0)),
            pl.BlockSpec((_HID, _C), lambda i: (0, 0)),
            pl.BlockSpec((1, _C), lambda i: (0, 0)),
            pl.BlockSpec((_PREP_BLK, 1), lambda i: (i, 0)),
        ],
        out_specs=[pl.BlockSpec((_NC, _PREP_BLK, _CH), lambda i: (0, i, 0))] * 5,
        out_shape=[shp] * 5,
    )(x_p, W1, b1, W2, b2, deg2)


@functools.partial(
    pl.kernel,
    out_type=jax.ShapeDtypeStruct((_NC, _NPAD, _CH), jnp.float32),
    mesh=_mesh2,
    compiler_params=_sc_params,
    scratch_types=[
        pltpu.VMEM_SHARED((_NPAD, _CH), jnp.float32),
        pltpu.VMEM_SHARED((_NPAD, _CH), jnp.float32),
        pltpu.VMEM((_NCH, _CHUNK), jnp.int32),
        pltpu.VMEM((_NB, 2 * _CHUNK), jnp.int32),
        pltpu.VMEM((_NB, 2 * _CHUNK), jnp.int32),
        pltpu.VMEM((_NB, 2 * _CHUNK, _CH), jnp.float32),
        pltpu.SemaphoreType.DMA((_NB,)),
        pltpu.SemaphoreType.DMA((_NB,)),
        pltpu.VMEM((2, _UCH, _CH), jnp.float32),
        pltpu.VMEM((2, _UCH, _CH), jnp.float32),
        pltpu.VMEM((2, _UCH, _CH), jnp.float32),
        pltpu.VMEM((2, _UCH, _CH), jnp.float32),
        pltpu.SemaphoreType.DMA((2,)),
        pltpu.SemaphoreType.DMA((2,)),
        pltpu.SemaphoreType.DMA((2,)),
        pltpu.SemaphoreType.DMA((2,)),
        pltpu.SemaphoreType.DMA((2,)),
    ],
)
def _prop_kernel(pidx_hbm, a_hbm, b_hbm, a2_hbm, b2_hbm, q0_hbm,
                 out_hbm, q_sh, s_sh, pidx, srcu, dstu, gbuf, gsem, ssem,
                 uS, uA, uB, uQ, lsemS, lsemA, lsemB, qsem, psem):
    cid = lax.axis_index("c")
    tid = lax.axis_index("s")
    row0 = tid * _ROWS_PT

    pltpu.sync_copy(pidx_hbm.at[tid], pidx)

    def initc(c, carry):
        r = row0 + c * _UCH
        pltpu.sync_copy(q0_hbm.at[cid, pl.ds(r, _UCH)], uQ.at[0])
        pltpu.sync_copy(uQ.at[0], q_sh.at[pl.ds(r, _UCH)])
        pltpu.sync_copy(uQ.at[0], s_sh.at[pl.ds(r, _UCH)])
        return carry

    lax.fori_loop(0, _UPC, initc, 0)
    plsc.subcore_barrier()

    def unpack(j, brow):
        for h in range(2):
            for v in range(_CHUNK // 16):
                sl = pl.ds(v * 16, 16)
                osl = pl.ds(h * _CHUNK + v * 16, 16)
                p = pidx[2 * j + h, sl]
                srcu[brow, osl] = lax.shift_right_logical(p, 14)
                dstu[brow, osl] = lax.bitwise_and(p, 16383)

    def edge_pass():
        unpack(0, 0)
        pltpu.async_copy(q_sh.at[srcu.at[0]], gbuf.at[0], gsem.at[0])

        def edge(j, carry):
            b = lax.rem(j, _NB)
            nb = lax.rem(j + 1, _NB)
            pltpu.make_async_copy(
                q_sh.at[srcu.at[b]], gbuf.at[b], gsem.at[b]).wait()
            pltpu.async_copy(gbuf.at[b], s_sh.at[dstu.at[b]], ssem.at[b],
                             add=True)

            @pl.when(j < _NCH // 2 - 1)
            def _():
                @pl.when(j >= _NB - 1)
                def _():
                    pltpu.make_async_copy(
                        gbuf.at[nb], s_sh.at[dstu.at[0]], ssem.at[nb]).wait()

                unpack(j + 1, nb)
                pltpu.async_copy(
                    q_sh.at[srcu.at[nb]], gbuf.at[nb], gsem.at[nb])

            return carry

        lax.fori_loop(0, _NCH // 2, edge, 0)
        for jj in range(_NCH // 2 - _NB, _NCH // 2):
            bb = jj % _NB
            pltpu.make_async_copy(
                gbuf.at[bb], s_sh.at[dstu.at[0]], ssem.at[bb]).wait()
        plsc.subcore_barrier()

    def update(a_src, b_src, to_hbm):
        def fire_loads(c, pb):
            r = row0 + c * _UCH
            pltpu.async_copy(s_sh.at[pl.ds(r, _UCH)], uS.at[pb], lsemS.at[pb])
            pltpu.async_copy(a_src.at[cid, pl.ds(r, _UCH)], uA.at[pb],
                             lsemA.at[pb])
            pltpu.async_copy(b_src.at[cid, pl.ds(r, _UCH)], uB.at[pb],
                             lsemB.at[pb])

        def wait_stores(pb):
            if to_hbm:
                pltpu.make_async_copy(
                    uQ.at[pb], out_hbm.at[cid, pl.ds(row0, _UCH)],
                    qsem.at[pb]).wait()
            else:
                pltpu.make_async_copy(
                    uQ.at[pb], q_sh.at[pl.ds(row0, _UCH)], qsem.at[pb]).wait()
                pltpu.make_async_copy(
                    uQ.at[pb], s_sh.at[pl.ds(row0, _UCH)], psem.at[pb]).wait()

        fire_loads(0, 0)

        def upd(c, carry):
            pb = lax.rem(c, 2)
            npb = 1 - pb
            r = row0 + c * _UCH

            @pl.when(c < _UPC - 1)
            def _():
                fire_loads(c + 1, npb)

            pltpu.make_async_copy(
                s_sh.at[pl.ds(row0, _UCH)], uS.at[pb], lsemS.at[pb]).wait()
            pltpu.make_async_copy(
                a_src.at[cid, pl.ds(row0, _UCH)], uA.at[pb],
                lsemA.at[pb]).wait()
            pltpu.make_async_copy(
                b_src.at[cid, pl.ds(row0, _UCH)], uB.at[pb],
                lsemB.at[pb]).wait()

            @pl.when(c >= 2)
            def _():
                wait_stores(pb)

            def rows(v, cc):
                for c3 in range(_CH // 16):
                    sl = pl.ds(c3 * 16, 16)
                    uQ[pb, v, sl] = uA[pb, v, sl] * uS[pb, v, sl] + uB[pb, v, sl]
                return cc

            lax.fori_loop(0, _UCH, rows, 0)
            if to_hbm:
                pltpu.async_copy(uQ.at[pb], out_hbm.at[cid, pl.ds(r, _UCH)],
                                 qsem.at[pb])
            else:
                pltpu.async_copy(uQ.at[pb], q_sh.at[pl.ds(r, _UCH)],
                                 qsem.at[pb])
                pltpu.async_copy(uQ.at[pb], s_sh.at[pl.ds(r, _UCH)],
                                 psem.at[pb])
            return carry

        lax.fori_loop(0, _UPC, upd, 0)
        for cc in (_UPC - 2, _UPC - 1):
            wait_stores(cc % 2)
        plsc.subcore_barrier()

    def step(k, carry):
        edge_pass()
        update(a_hbm, b_hbm, to_hbm=False)
        return carry

    lax.fori_loop(0, _K - 1, step, 0)
    edge_pass()
    update(a2_hbm, b2_hbm, to_hbm=True)


@jax.jit
def kernel(x, edge_index, W1, b1, W2, b2):
    src = edge_index[0]
    dst = edge_index[1]
    npad_ids = (jnp.arange(_EPAD - _E, dtype=jnp.int32) % (_NPAD - _N)) + _N
    src_p = jnp.concatenate([src, npad_ids])
    dst_p = jnp.concatenate([dst, npad_ids])
    dst_r = dst_p.reshape(_NT, _NCH, _CHUNK)
    pidx_r = (src_p * 16384 + dst_p).reshape(_NT, _NCH, _CHUNK)

    deg = _deg_kernel(dst_r)

    x_p = jnp.concatenate(
        [x, jnp.zeros((_NPAD - _N, _F), jnp.float32)], axis=0)
    A, B, A2, B2, Q0 = _prep_call(
        x_p, W1, b1.reshape(1, _HID), W2, b2.reshape(1, _C),
        deg.reshape(_NPAD, 1))

    out2 = _prop_kernel(pidx_r, A, B, A2, B2, Q0)
    return jnp.concatenate([out2[0], out2[1]], axis=1)[:_N, :_C]


# speedup vs baseline: 2.0376x; 1.0017x over previous
import functools

import jax
import jax.numpy as jnp
from jax import lax
from jax.experimental import pallas as pl
from jax.experimental.pallas import tpu as pltpu
from jax.experimental.pallas import tpu_sc as plsc

_N = 10000
_E = 320000
_F = 128
_HID = 64
_C = 40
_K = 10
_ALPHA = 0.1

_NT = 16
_CHUNK = 128
_GSZ = 16
_NGRP = 10
_NCH = _GSZ * _NGRP
_EPT = _NCH * _CHUNK
_EPAD = _NT * _EPT
_NPAD = 10240
_CPAD = 48
_CP2 = 64
_CH = 32
_ROWS_PT = _NPAD // _NT
_UCH = 64
_NB = 6
_UPC = _ROWS_PT // _UCH
_NC = 2

_mesh = plsc.VectorSubcoreMesh(
    core_axis_name="c", subcore_axis_name="s", num_cores=1
)
_mesh2 = plsc.VectorSubcoreMesh(
    core_axis_name="c", subcore_axis_name="s", num_cores=2
)
_sc_params = pltpu.CompilerParams(use_tc_tiling_on_sc=False)


@functools.partial(
    pl.kernel,
    out_type=jax.ShapeDtypeStruct((_NPAD,), jnp.float32),
    mesh=_mesh,
    compiler_params=_sc_params,
    scratch_types=[
        pltpu.VMEM_SHARED((_NPAD,), jnp.float32),
        pltpu.VMEM((_GSZ, _CHUNK), jnp.int32),
        pltpu.VMEM((_ROWS_PT,), jnp.float32),
    ],
)
def _deg_kernel(dst_hbm, deg_hbm, deg_sh, didx, ones):
    tid = lax.axis_index("s")
    row0 = tid * _ROWS_PT

    def fill(i, c):
        ones[pl.ds(i * 16, 16)] = jnp.full((16,), 1.0, jnp.float32)
        return c

    lax.fori_loop(0, _ROWS_PT // 16, fill, 0)
    pltpu.sync_copy(ones, deg_sh.at[pl.ds(row0, _ROWS_PT)])
    plsc.subcore_barrier()

    def group(g, c):
        pltpu.sync_copy(dst_hbm.at[tid, pl.ds(g * _GSZ, _GSZ)], didx)

        def chunk(j, cc):
            pltpu.sync_copy(
                ones.at[pl.ds(0, _CHUNK)], deg_sh.at[didx.at[j]], add=True)
            return cc

        lax.fori_loop(0, _GSZ, chunk, 0)
        return c

    lax.fori_loop(0, _NGRP, group, 0)
    plsc.subcore_barrier()
    pltpu.sync_copy(deg_sh.at[pl.ds(row0, _ROWS_PT)], deg_hbm.at[pl.ds(row0, _ROWS_PT)])


def _prep_body(x_ref, w1_ref, b1_ref, w2_ref, b2_ref, deg_ref,
               a_ref, b_ref, a2_ref, b2_ref_o, q0_ref):
    h = jnp.maximum(
        jax.lax.dot_general(
            x_ref[...], w1_ref[...], (((1,), (0,)), ((), ())),
            preferred_element_type=jnp.float32,
        ) + b1_ref[...], 0.0)
    h = jax.lax.dot_general(
        h, w2_ref[...], (((1,), (0,)), ((), ())),
        preferred_element_type=jnp.float32,
    ) + b2_ref[...]
    blk = h.shape[0]
    hp = jnp.concatenate([h, jnp.zeros((blk, _CP2 - _C), jnp.float32)], axis=1)
    dinv = jax.lax.rsqrt(deg_ref[...])

    def split(arr):
        return jnp.stack([arr[:, :_CH], arr[:, _CH:]], axis=0)

    a_ref[...] = split(jnp.broadcast_to((1.0 - _ALPHA) * dinv * dinv,
                                        (blk, _CP2)))
    a2_ref[...] = split(jnp.broadcast_to((1.0 - _ALPHA) * dinv, (blk, _CP2)))
    b_ref[...] = split(_ALPHA * dinv * hp)
    b2_ref_o[...] = split(_ALPHA * hp)
    q0_ref[...] = split(dinv * hp)


_PREP_BLK = 2048


def _prep_call(x_p, W1, b1, W2, b2, deg2):
    grid = (_NPAD // _PREP_BLK,)
    shp = jax.ShapeDtypeStruct((_NC, _NPAD, _CH), jnp.float32)
    return pl.pallas_call(
        _prep_body,
        grid=grid,
        in_specs=[
            pl.BlockSpec((_PREP_BLK, _F), lambda i: (i, 0)),
            pl.BlockSpec((_F, _HID), lambda i: (0, 0)),
            pl.BlockSpec((1, _HID), lambda i: (0, 0)),
            pl.BlockSpec((_HID, _C), lambda i: (0, 0)),
            pl.BlockSpec((1, _C), lambda i: (0, 0)),
            pl.BlockSpec((_PREP_BLK, 1), lambda i: (i, 0)),
        ],
        out_specs=[pl.BlockSpec((_NC, _PREP_BLK, _CH), lambda i: (0, i, 0))] * 5,
        out_shape=[shp] * 5,
    )(x_p, W1, b1, W2, b2, deg2)


@functools.partial(
    pl.kernel,
    out_type=jax.ShapeDtypeStruct((_NC, _NPAD, _CH), jnp.float32),
    mesh=_mesh2,
    compiler_params=_sc_params,
    scratch_types=[
        pltpu.VMEM_SHARED((_NPAD, _CH), jnp.float32),
        pltpu.VMEM_SHARED((_NPAD, _CH), jnp.float32),
        pltpu.VMEM((_NCH, _CHUNK), jnp.int32),
        pltpu.VMEM((_NB, 2 * _CHUNK), jnp.int32),
        pltpu.VMEM((_NB, 2 * _CHUNK), jnp.int32),
        pltpu.VMEM((_NB, 2 * _CHUNK, _CH), jnp.float32),
        pltpu.SemaphoreType.DMA((_NB,)),
        pltpu.SemaphoreType.DMA((_NB,)),
        pltpu.VMEM((2, _UCH, _CH), jnp.float32),
        pltpu.VMEM((2, _UCH, _CH), jnp.float32),
        pltpu.VMEM((2, _UCH, _CH), jnp.float32),
        pltpu.VMEM((2, _UCH, _CH), jnp.float32),
        pltpu.SemaphoreType.DMA((2,)),
        pltpu.SemaphoreType.DMA((2,)),
        pltpu.SemaphoreType.DMA((2,)),
        pltpu.SemaphoreType.DMA((2,)),
        pltpu.SemaphoreType.DMA((2,)),
    ],
)
def _prop_kernel(pidx_hbm, a_hbm, b_hbm, a2_hbm, b2_hbm, q0_hbm,
                 out_hbm, q_sh, s_sh, pidx, srcu, dstu, gbuf, gsem, ssem,
                 uS, uA, uB, uQ, lsemS, lsemA, lsemB, qsem, psem):
    cid = lax.axis_index("c")
    tid = lax.axis_index("s")
    row0 = tid * _ROWS_PT

    pltpu.sync_copy(pidx_hbm.at[tid], pidx)

    def initc(c, carry):
        r = row0 + c * _UCH
        pltpu.sync_copy(q0_hbm.at[cid, pl.ds(r, _UCH)], uQ.at[0])
        pltpu.sync_copy(uQ.at[0], q_sh.at[pl.ds(r, _UCH)])
        pltpu.sync_copy(uQ.at[0], s_sh.at[pl.ds(r, _UCH)])
        return carry

    lax.fori_loop(0, _UPC, initc, 0)
    plsc.subcore_barrier()

    def unpack(j, brow):
        for h in range(2):
            for v in range(_CHUNK // 16):
                sl = pl.ds(v * 16, 16)
                osl = pl.ds(h * _CHUNK + v * 16, 16)
                p = pidx[2 * j + h, sl]
                srcu[brow, osl] = lax.shift_right_logical(p, 14)
                dstu[brow, osl] = lax.bitwise_and(p, 16383)

    def edge_pass():
        unpack(0, 0)
        pltpu.async_copy(q_sh.at[srcu.at[0]], gbuf.at[0], gsem.at[0])

        def edge(j, carry):
            b = lax.rem(j, _NB)
            nb = lax.rem(j + 1, _NB)
            pltpu.make_async_copy(
                q_sh.at[srcu.at[b]], gbuf.at[b], gsem.at[b]).wait()
            pltpu.async_copy(gbuf.at[b], s_sh.at[dstu.at[b]], ssem.at[b],
                             add=True)

            @pl.when(j < _NCH // 2 - 1)
            def _():
                @pl.when(j >= _NB - 1)
                def _():
                    pltpu.make_async_copy(
                        gbuf.at[nb], s_sh.at[dstu.at[0]], ssem.at[nb]).wait()

                unpack(j + 1, nb)
                pltpu.async_copy(
                    q_sh.at[srcu.at[nb]], gbuf.at[nb], gsem.at[nb])

            return carry

        lax.fori_loop(0, _NCH // 2, edge, 0)
        for jj in range(_NCH // 2 - _NB, _NCH // 2):
            bb = jj % _NB
            pltpu.make_async_copy(
                gbuf.at[bb], s_sh.at[dstu.at[0]], ssem.at[bb]).wait()
        plsc.subcore_barrier()

    def update(a_src, b_src, to_hbm):
        def fire_loads(c, pb):
            r = row0 + c * _UCH
            pltpu.async_copy(s_sh.at[pl.ds(r, _UCH)], uS.at[pb], lsemS.at[pb])
            pltpu.async_copy(a_src.at[cid, pl.ds(r, _UCH)], uA.at[pb],
                             lsemA.at[pb])
            pltpu.async_copy(b_src.at[cid, pl.ds(r, _UCH)], uB.at[pb],
                             lsemB.at[pb])

        def wait_stores(pb):
            if to_hbm:
                pltpu.make_async_copy(
                    uQ.at[pb], out_hbm.at[cid, pl.ds(row0, _UCH)],
                    qsem.at[pb]).wait()
            else:
                pltpu.make_async_copy(
                    uQ.at[pb], q_sh.at[pl.ds(row0, _UCH)], qsem.at[pb]).wait()
                pltpu.make_async_copy(
                    uQ.at[pb], s_sh.at[pl.ds(row0, _UCH)], psem.at[pb]).wait()

        fire_loads(0, 0)

        def upd(c, carry):
            pb = lax.rem(c, 2)
            npb = 1 - pb
            r = row0 + c * _UCH

            @pl.when(c < _UPC - 1)
            def _():
                fire_loads(c + 1, npb)

            pltpu.make_async_copy(
                s_sh.at[pl.ds(row0, _UCH)], uS.at[pb], lsemS.at[pb]).wait()
            pltpu.make_async_copy(
                a_src.at[cid, pl.ds(row0, _UCH)], uA.at[pb],
                lsemA.at[pb]).wait()
            pltpu.make_async_copy(
                b_src.at[cid, pl.ds(row0, _UCH)], uB.at[pb],
                lsemB.at[pb]).wait()

            @pl.when(c >= 2)
            def _():
                wait_stores(pb)

            def rows(v, cc):
                for c3 in range(_CH // 16):
                    sl = pl.ds(c3 * 16, 16)
                    uQ[pb, v, sl] = uA[pb, v, sl] * uS[pb, v, sl] + uB[pb, v, sl]
                return cc

            lax.fori_loop(0, _UCH, rows, 0)
            if to_hbm:
                pltpu.async_copy(uQ.at[pb], out_hbm.at[cid, pl.ds(r, _UCH)],
                                 qsem.at[pb])
            else:
                pltpu.async_copy(uQ.at[pb], q_sh.at[pl.ds(r, _UCH)],
                                 qsem.at[pb])
                pltpu.async_copy(uQ.at[pb], s_sh.at[pl.ds(r, _UCH)],
                                 psem.at[pb])
            return carry

        lax.fori_loop(0, _UPC, upd, 0)
        for cc in (_UPC - 2, _UPC - 1):
            wait_stores(cc % 2)
        plsc.subcore_barrier()

    def step(k, carry):
        edge_pass()
        update(a_hbm, b_hbm, to_hbm=False)
        return carry

    lax.fori_loop(0, _K - 1, step, 0)
    edge_pass()
    update(a2_hbm, b2_hbm, to_hbm=True)


@jax.jit
def kernel(x, edge_index, W1, b1, W2, b2):
    src = edge_index[0]
    dst = edge_index[1]
    npad_ids = (jnp.arange(_EPAD - _E, dtype=jnp.int32) % (_NPAD - _N)) + _N
    src_p = jnp.concatenate([src, npad_ids])
    dst_p = jnp.concatenate([dst, npad_ids])
    dst_r = dst_p.reshape(_NT, _NCH, _CHUNK)
    pidx_r = (src_p * 16384 + dst_p).reshape(_NT, _NCH, _CHUNK)

    deg = _deg_kernel(dst_r)

    x_p = jnp.concatenate(
        [x, jnp.zeros((_NPAD - _N, _F), jnp.float32)], axis=0)
    A, B, A2, B2, Q0 = _prep_call(
        x_p, W1, b1.reshape(1, _HID), W2, b2.reshape(1, _C),
        deg.reshape(_NPAD, 1))

    out2 = _prop_kernel(pidx_r, A, B, A2, B2, Q0)
    return jnp.concatenate([out2[0], out2[1]], axis=1)[:_N, :_C]


# speedup vs baseline: 2.1161x; 1.0385x over previous
import functools

import jax
import jax.numpy as jnp
from jax import lax
from jax.experimental import pallas as pl
from jax.experimental.pallas import tpu as pltpu
from jax.experimental.pallas import tpu_sc as plsc

_N = 10000
_E = 320000
_F = 128
_HID = 64
_C = 40
_K = 10
_ALPHA = 0.1

_NT = 16
_CHUNK = 128
_GSZ = 16
_NGRP = 10
_NCH = _GSZ * _NGRP
_EPT = _NCH * _CHUNK
_EPAD = _NT * _EPT
_NPAD = 10240
_CPAD = 48
_CP2 = 64
_CH = 32
_ROWS_PT = _NPAD // _NT
_UCH = 64
_NB = 3
_EM = 4
_UPC = _ROWS_PT // _UCH
_NC = 2

_mesh = plsc.VectorSubcoreMesh(
    core_axis_name="c", subcore_axis_name="s", num_cores=1
)
_mesh2 = plsc.VectorSubcoreMesh(
    core_axis_name="c", subcore_axis_name="s", num_cores=2
)
_sc_params = pltpu.CompilerParams(use_tc_tiling_on_sc=False)


@functools.partial(
    pl.kernel,
    out_type=jax.ShapeDtypeStruct((_NPAD,), jnp.float32),
    mesh=_mesh,
    compiler_params=_sc_params,
    scratch_types=[
        pltpu.VMEM_SHARED((_NPAD,), jnp.float32),
        pltpu.VMEM((_GSZ, _CHUNK), jnp.int32),
        pltpu.VMEM((_ROWS_PT,), jnp.float32),
    ],
)
def _deg_kernel(dst_hbm, deg_hbm, deg_sh, didx, ones):
    tid = lax.axis_index("s")
    row0 = tid * _ROWS_PT

    def fill(i, c):
        ones[pl.ds(i * 16, 16)] = jnp.full((16,), 1.0, jnp.float32)
        return c

    lax.fori_loop(0, _ROWS_PT // 16, fill, 0)
    pltpu.sync_copy(ones, deg_sh.at[pl.ds(row0, _ROWS_PT)])
    plsc.subcore_barrier()

    def group(g, c):
        pltpu.sync_copy(dst_hbm.at[tid, pl.ds(g * _GSZ, _GSZ)], didx)

        def chunk(j, cc):
            pltpu.sync_copy(
                ones.at[pl.ds(0, _CHUNK)], deg_sh.at[didx.at[j]], add=True)
            return cc

        lax.fori_loop(0, _GSZ, chunk, 0)
        return c

    lax.fori_loop(0, _NGRP, group, 0)
    plsc.subcore_barrier()
    pltpu.sync_copy(deg_sh.at[pl.ds(row0, _ROWS_PT)], deg_hbm.at[pl.ds(row0, _ROWS_PT)])


def _prep_body(x_ref, w1_ref, b1_ref, w2_ref, b2_ref, deg_ref,
               a_ref, b_ref, a2_ref, b2_ref_o, q0_ref):
    h = jnp.maximum(
        jax.lax.dot_general(
            x_ref[...], w1_ref[...], (((1,), (0,)), ((), ())),
            preferred_element_type=jnp.float32,
        ) + b1_ref[...], 0.0)
    h = jax.lax.dot_general(
        h, w2_ref[...], (((1,), (0,)), ((), ())),
        preferred_element_type=jnp.float32,
    ) + b2_ref[...]
    blk = h.shape[0]
    hp = jnp.concatenate([h, jnp.zeros((blk, _CP2 - _C), jnp.float32)], axis=1)
    dinv = jax.lax.rsqrt(deg_ref[...])

    def split(arr):
        return jnp.stack([arr[:, :_CH], arr[:, _CH:]], axis=0)

    a_ref[...] = split(jnp.broadcast_to((1.0 - _ALPHA) * dinv * dinv,
                                        (blk, _CP2)))
    a2_ref[...] = split(jnp.broadcast_to((1.0 - _ALPHA) * dinv, (blk, _CP2)))
    b_ref[...] = split(_ALPHA * dinv * hp)
    b2_ref_o[...] = split(_ALPHA * hp)
    q0_ref[...] = split(dinv * hp)


_PREP_BLK = 2048


def _prep_call(x_p, W1, b1, W2, b2, deg2):
    grid = (_NPAD // _PREP_BLK,)
    shp = jax.ShapeDtypeStruct((_NC, _NPAD, _CH), jnp.float32)
    return pl.pallas_call(
        _prep_body,
        grid=grid,
        in_specs=[
            pl.BlockSpec((_PREP_BLK, _F), lambda i: (i, 0)),
            pl.BlockSpec((_F, _HID), lambda i: (0, 0)),
            pl.BlockSpec((1, _HID), lambda i: (0, 0)),
            pl.BlockSpec((_HID, _C), lambda i: (0, 0)),
            pl.BlockSpec((1, _C), lambda i: (0, 0)),
            pl.BlockSpec((_PREP_BLK, 1), lambda i: (i, 0)),
        ],
        out_specs=[pl.BlockSpec((_NC, _PREP_BLK, _CH), lambda i: (0, i, 0))] * 5,
        out_shape=[shp] * 5,
    )(x_p, W1, b1, W2, b2, deg2)


@functools.partial(
    pl.kernel,
    out_type=jax.ShapeDtypeStruct((_NC, _NPAD, _CH), jnp.float32),
    mesh=_mesh2,
    compiler_params=_sc_params,
    scratch_types=[
        pltpu.VMEM_SHARED((_NPAD, _CH), jnp.float32),
        pltpu.VMEM_SHARED((_NPAD, _CH), jnp.float32),
        pltpu.VMEM((_NCH, _CHUNK), jnp.int32),
        pltpu.VMEM((_NB, _EM * _CHUNK), jnp.int32),
        pltpu.VMEM((_NB, _EM * _CHUNK), jnp.int32),
        pltpu.VMEM((_NB, _EM * _CHUNK, _CH), jnp.float32),
        pltpu.SemaphoreType.DMA((_NB,)),
        pltpu.SemaphoreType.DMA((_NB,)),
        pltpu.VMEM((2, _UCH, _CH), jnp.float32),
        pltpu.VMEM((2, _UCH, _CH), jnp.float32),
        pltpu.VMEM((2, _UCH, _CH), jnp.float32),
        pltpu.VMEM((2, _UCH, _CH), jnp.float32),
        pltpu.SemaphoreType.DMA((2,)),
        pltpu.SemaphoreType.DMA((2,)),
        pltpu.SemaphoreType.DMA((2,)),
        pltpu.SemaphoreType.DMA((2,)),
        pltpu.SemaphoreType.DMA((2,)),
    ],
)
def _prop_kernel(pidx_hbm, a_hbm, b_hbm, a2_hbm, b2_hbm, q0_hbm,
                 out_hbm, q_sh, s_sh, pidx, srcu, dstu, gbuf, gsem, ssem,
                 uS, uA, uB, uQ, lsemS, lsemA, lsemB, qsem, psem):
    cid = lax.axis_index("c")
    tid = lax.axis_index("s")
    row0 = tid * _ROWS_PT

    pltpu.sync_copy(pidx_hbm.at[tid], pidx)

    def initc(c, carry):
        r = row0 + c * _UCH
        pltpu.sync_copy(q0_hbm.at[cid, pl.ds(r, _UCH)], uQ.at[0])
        pltpu.sync_copy(uQ.at[0], q_sh.at[pl.ds(r, _UCH)])
        pltpu.sync_copy(uQ.at[0], s_sh.at[pl.ds(r, _UCH)])
        return carry

    lax.fori_loop(0, _UPC, initc, 0)
    plsc.subcore_barrier()

    def unpack(j, brow):
        for h in range(_EM):
            for v in range(_CHUNK // 16):
                sl = pl.ds(v * 16, 16)
                osl = pl.ds(h * _CHUNK + v * 16, 16)
                p = pidx[_EM * j + h, sl]
                srcu[brow, osl] = lax.shift_right_logical(p, 14)
                dstu[brow, osl] = lax.bitwise_and(p, 16383)

    def edge_pass():
        unpack(0, 0)
        pltpu.async_copy(q_sh.at[srcu.at[0]], gbuf.at[0], gsem.at[0])

        def edge(j, carry):
            b = lax.rem(j, _NB)
            nb = lax.rem(j + 1, _NB)
            pltpu.make_async_copy(
                q_sh.at[srcu.at[b]], gbuf.at[b], gsem.at[b]).wait()
            pltpu.async_copy(gbuf.at[b], s_sh.at[dstu.at[b]], ssem.at[b],
                             add=True)

            @pl.when(j < _NCH // _EM - 1)
            def _():
                @pl.when(j >= _NB - 1)
                def _():
                    pltpu.make_async_copy(
                        gbuf.at[nb], s_sh.at[dstu.at[0]], ssem.at[nb]).wait()

                unpack(j + 1, nb)
                pltpu.async_copy(
                    q_sh.at[srcu.at[nb]], gbuf.at[nb], gsem.at[nb])

            return carry

        lax.fori_loop(0, _NCH // _EM, edge, 0)
        for jj in range(_NCH // _EM - _NB, _NCH // _EM):
            bb = jj % _NB
            pltpu.make_async_copy(
                gbuf.at[bb], s_sh.at[dstu.at[0]], ssem.at[bb]).wait()
        plsc.subcore_barrier()

    def update(a_src, b_src, to_hbm):
        def fire_loads(c, pb):
            r = row0 + c * _UCH
            pltpu.async_copy(s_sh.at[pl.ds(r, _UCH)], uS.at[pb], lsemS.at[pb])
            pltpu.async_copy(a_src.at[cid, pl.ds(r, _UCH)], uA.at[pb],
                             lsemA.at[pb])
            pltpu.async_copy(b_src.at[cid, pl.ds(r, _UCH)], uB.at[pb],
                             lsemB.at[pb])

        def wait_stores(pb):
            if to_hbm:
                pltpu.make_async_copy(
                    uQ.at[pb], out_hbm.at[cid, pl.ds(row0, _UCH)],
                    qsem.at[pb]).wait()
            else:
                pltpu.make_async_copy(
                    uQ.at[pb], q_sh.at[pl.ds(row0, _UCH)], qsem.at[pb]).wait()
                pltpu.make_async_copy(
                    uQ.at[pb], s_sh.at[pl.ds(row0, _UCH)], psem.at[pb]).wait()

        fire_loads(0, 0)

        def upd(c, carry):
            pb = lax.rem(c, 2)
            npb = 1 - pb
            r = row0 + c * _UCH

            @pl.when(c < _UPC - 1)
            def _():
                fire_loads(c + 1, npb)

            pltpu.make_async_copy(
                s_sh.at[pl.ds(row0, _UCH)], uS.at[pb], lsemS.at[pb]).wait()
            pltpu.make_async_copy(
                a_src.at[cid, pl.ds(row0, _UCH)], uA.at[pb],
                lsemA.at[pb]).wait()
            pltpu.make_async_copy(
                b_src.at[cid, pl.ds(row0, _UCH)], uB.at[pb],
                lsemB.at[pb]).wait()

            @pl.when(c >= 2)
            def _():
                wait_stores(pb)

            def rows(v, cc):
                for c3 in range(_CH // 16):
                    sl = pl.ds(c3 * 16, 16)
                    uQ[pb, v, sl] = uA[pb, v, sl] * uS[pb, v, sl] + uB[pb, v, sl]
                return cc

            lax.fori_loop(0, _UCH, rows, 0)
            if to_hbm:
                pltpu.async_copy(uQ.at[pb], out_hbm.at[cid, pl.ds(r, _UCH)],
                                 qsem.at[pb])
            else:
                pltpu.async_copy(uQ.at[pb], q_sh.at[pl.ds(r, _UCH)],
                                 qsem.at[pb])
                pltpu.async_copy(uQ.at[pb], s_sh.at[pl.ds(r, _UCH)],
                                 psem.at[pb])
            return carry

        lax.fori_loop(0, _UPC, upd, 0)
        for cc in (_UPC - 2, _UPC - 1):
            wait_stores(cc % 2)
        plsc.subcore_barrier()

    def step(k, carry):
        edge_pass()
        update(a_hbm, b_hbm, to_hbm=False)
        return carry

    lax.fori_loop(0, _K - 1, step, 0)
    edge_pass()
    update(a2_hbm, b2_hbm, to_hbm=True)


@jax.jit
def kernel(x, edge_index, W1, b1, W2, b2):
    src = edge_index[0]
    dst = edge_index[1]
    npad_ids = (jnp.arange(_EPAD - _E, dtype=jnp.int32) % (_NPAD - _N)) + _N
    src_p = jnp.concatenate([src, npad_ids])
    dst_p = jnp.concatenate([dst, npad_ids])
    dst_r = dst_p.reshape(_NT, _NCH, _CHUNK)
    pidx_r = (src_p * 16384 + dst_p).reshape(_NT, _NCH, _CHUNK)

    deg = _deg_kernel(dst_r)

    x_p = jnp.concatenate(
        [x, jnp.zeros((_NPAD - _N, _F), jnp.float32)], axis=0)
    A, B, A2, B2, Q0 = _prep_call(
        x_p, W1, b1.reshape(1, _HID), W2, b2.reshape(1, _C),
        deg.reshape(_NPAD, 1))

    out2 = _prop_kernel(pidx_r, A, B, A2, B2, Q0)
    return jnp.concatenate([out2[0], out2[1]], axis=1)[:_N, :_C]


# speedup vs baseline: 2.1384x; 1.0105x over previous
import functools

import jax
import jax.numpy as jnp
from jax import lax
from jax.experimental import pallas as pl
from jax.experimental.pallas import tpu as pltpu
from jax.experimental.pallas import tpu_sc as plsc

_N = 10000
_E = 320000
_F = 128
_HID = 64
_C = 40
_K = 10
_ALPHA = 0.1

_NT = 16
_CHUNK = 128
_GSZ = 16
_NGRP = 10
_NCH = _GSZ * _NGRP
_EPT = _NCH * _CHUNK
_EPAD = _NT * _EPT
_NPAD = 10240
_CPAD = 48
_CP2 = 64
_CH = 32
_ROWS_PT = _NPAD // _NT
_UCH = 64
_NB = 2
_EM = 5
_UPC = _ROWS_PT // _UCH
_NC = 2

_mesh = plsc.VectorSubcoreMesh(
    core_axis_name="c", subcore_axis_name="s", num_cores=1
)
_mesh2 = plsc.VectorSubcoreMesh(
    core_axis_name="c", subcore_axis_name="s", num_cores=2
)
_sc_params = pltpu.CompilerParams(use_tc_tiling_on_sc=False)


@functools.partial(
    pl.kernel,
    out_type=jax.ShapeDtypeStruct((_NPAD,), jnp.float32),
    mesh=_mesh,
    compiler_params=_sc_params,
    scratch_types=[
        pltpu.VMEM_SHARED((_NPAD,), jnp.float32),
        pltpu.VMEM((_GSZ, _CHUNK), jnp.int32),
        pltpu.VMEM((_ROWS_PT,), jnp.float32),
    ],
)
def _deg_kernel(dst_hbm, deg_hbm, deg_sh, didx, ones):
    tid = lax.axis_index("s")
    row0 = tid * _ROWS_PT

    def fill(i, c):
        ones[pl.ds(i * 16, 16)] = jnp.full((16,), 1.0, jnp.float32)
        return c

    lax.fori_loop(0, _ROWS_PT // 16, fill, 0)
    pltpu.sync_copy(ones, deg_sh.at[pl.ds(row0, _ROWS_PT)])
    plsc.subcore_barrier()

    def group(g, c):
        pltpu.sync_copy(dst_hbm.at[tid, pl.ds(g * _GSZ, _GSZ)], didx)

        def chunk(j, cc):
            pltpu.sync_copy(
                ones.at[pl.ds(0, _CHUNK)], deg_sh.at[didx.at[j]], add=True)
            return cc

        lax.fori_loop(0, _GSZ, chunk, 0)
        return c

    lax.fori_loop(0, _NGRP, group, 0)
    plsc.subcore_barrier()
    pltpu.sync_copy(deg_sh.at[pl.ds(row0, _ROWS_PT)], deg_hbm.at[pl.ds(row0, _ROWS_PT)])


def _prep_body(x_ref, w1_ref, b1_ref, w2_ref, b2_ref, deg_ref,
               a_ref, b_ref, a2_ref, b2_ref_o, q0_ref):
    h = jnp.maximum(
        jax.lax.dot_general(
            x_ref[...], w1_ref[...], (((1,), (0,)), ((), ())),
            preferred_element_type=jnp.float32,
        ) + b1_ref[...], 0.0)
    h = jax.lax.dot_general(
        h, w2_ref[...], (((1,), (0,)), ((), ())),
        preferred_element_type=jnp.float32,
    ) + b2_ref[...]
    blk = h.shape[0]
    hp = jnp.concatenate([h, jnp.zeros((blk, _CP2 - _C), jnp.float32)], axis=1)
    dinv = jax.lax.rsqrt(deg_ref[...])

    def split(arr):
        return jnp.stack([arr[:, :_CH], arr[:, _CH:]], axis=0)

    a_ref[...] = split(jnp.broadcast_to((1.0 - _ALPHA) * dinv * dinv,
                                        (blk, _CP2)))
    a2_ref[...] = split(jnp.broadcast_to((1.0 - _ALPHA) * dinv, (blk, _CP2)))
    b_ref[...] = split(_ALPHA * dinv * hp)
    b2_ref_o[...] = split(_ALPHA * hp)
    q0_ref[...] = split(dinv * hp)


_PREP_BLK = 2048


def _prep_call(x_p, W1, b1, W2, b2, deg2):
    grid = (_NPAD // _PREP_BLK,)
    shp = jax.ShapeDtypeStruct((_NC, _NPAD, _CH), jnp.float32)
    return pl.pallas_call(
        _prep_body,
        grid=grid,
        in_specs=[
            pl.BlockSpec((_PREP_BLK, _F), lambda i: (i, 0)),
            pl.BlockSpec((_F, _HID), lambda i: (0, 0)),
            pl.BlockSpec((1, _HID), lambda i: (0, 0)),
            pl.BlockSpec((_HID, _C), lambda i: (0, 0)),
            pl.BlockSpec((1, _C), lambda i: (0, 0)),
            pl.BlockSpec((_PREP_BLK, 1), lambda i: (i, 0)),
        ],
        out_specs=[pl.BlockSpec((_NC, _PREP_BLK, _CH), lambda i: (0, i, 0))] * 5,
        out_shape=[shp] * 5,
    )(x_p, W1, b1, W2, b2, deg2)


@functools.partial(
    pl.kernel,
    out_type=jax.ShapeDtypeStruct((_NC, _NPAD, _CH), jnp.float32),
    mesh=_mesh2,
    compiler_params=_sc_params,
    scratch_types=[
        pltpu.VMEM_SHARED((_NPAD, _CH), jnp.float32),
        pltpu.VMEM_SHARED((_NPAD, _CH), jnp.float32),
        pltpu.VMEM((_NCH, _CHUNK), jnp.int32),
        pltpu.VMEM((_NB, _EM * _CHUNK), jnp.int32),
        pltpu.VMEM((_NB, _EM * _CHUNK), jnp.int32),
        pltpu.VMEM((_NB, _EM * _CHUNK, _CH), jnp.float32),
        pltpu.SemaphoreType.DMA((_NB,)),
        pltpu.SemaphoreType.DMA((_NB,)),
        pltpu.VMEM((2, _UCH, _CH), jnp.float32),
        pltpu.VMEM((2, _UCH, _CH), jnp.float32),
        pltpu.VMEM((2, _UCH, _CH), jnp.float32),
        pltpu.VMEM((2, _UCH, _CH), jnp.float32),
        pltpu.SemaphoreType.DMA((2,)),
        pltpu.SemaphoreType.DMA((2,)),
        pltpu.SemaphoreType.DMA((2,)),
        pltpu.SemaphoreType.DMA((2,)),
        pltpu.SemaphoreType.DMA((2,)),
    ],
)
def _prop_kernel(pidx_hbm, a_hbm, b_hbm, a2_hbm, b2_hbm, q0_hbm,
                 out_hbm, q_sh, s_sh, pidx, srcu, dstu, gbuf, gsem, ssem,
                 uS, uA, uB, uQ, lsemS, lsemA, lsemB, qsem, psem):
    cid = lax.axis_index("c")
    tid = lax.axis_index("s")
    row0 = tid * _ROWS_PT

    pltpu.sync_copy(pidx_hbm.at[tid], pidx)

    def initc(c, carry):
        r = row0 + c * _UCH
        pltpu.sync_copy(q0_hbm.at[cid, pl.ds(r, _UCH)], uQ.at[0])
        pltpu.sync_copy(uQ.at[0], q_sh.at[pl.ds(r, _UCH)])
        pltpu.sync_copy(uQ.at[0], s_sh.at[pl.ds(r, _UCH)])
        return carry

    lax.fori_loop(0, _UPC, initc, 0)
    plsc.subcore_barrier()

    def unpack(j, brow):
        for h in range(_EM):
            for v in range(_CHUNK // 16):
                sl = pl.ds(v * 16, 16)
                osl = pl.ds(h * _CHUNK + v * 16, 16)
                p = pidx[_EM * j + h, sl]
                srcu[brow, osl] = lax.shift_right_logical(p, 14)
                dstu[brow, osl] = lax.bitwise_and(p, 16383)

    def edge_pass():
        unpack(0, 0)
        pltpu.async_copy(q_sh.at[srcu.at[0]], gbuf.at[0], gsem.at[0])

        def edge(j, carry):
            b = lax.rem(j, _NB)
            nb = lax.rem(j + 1, _NB)
            pltpu.make_async_copy(
                q_sh.at[srcu.at[b]], gbuf.at[b], gsem.at[b]).wait()
            pltpu.async_copy(gbuf.at[b], s_sh.at[dstu.at[b]], ssem.at[b],
                             add=True)

            @pl.when(j < _NCH // _EM - 1)
            def _():
                @pl.when(j >= _NB - 1)
                def _():
                    pltpu.make_async_copy(
                        gbuf.at[nb], s_sh.at[dstu.at[0]], ssem.at[nb]).wait()

                unpack(j + 1, nb)
                pltpu.async_copy(
                    q_sh.at[srcu.at[nb]], gbuf.at[nb], gsem.at[nb])

            return carry

        lax.fori_loop(0, _NCH // _EM, edge, 0)
        for jj in range(_NCH // _EM - _NB, _NCH // _EM):
            bb = jj % _NB
            pltpu.make_async_copy(
                gbuf.at[bb], s_sh.at[dstu.at[0]], ssem.at[bb]).wait()
        plsc.subcore_barrier()

    def update(a_src, b_src, to_hbm):
        def fire_loads(c, pb):
            r = row0 + c * _UCH
            pltpu.async_copy(s_sh.at[pl.ds(r, _UCH)], uS.at[pb], lsemS.at[pb])
            pltpu.async_copy(a_src.at[cid, pl.ds(r, _UCH)], uA.at[pb],
                             lsemA.at[pb])
            pltpu.async_copy(b_src.at[cid, pl.ds(r, _UCH)], uB.at[pb],
                             lsemB.at[pb])

        def wait_stores(pb):
            if to_hbm:
                pltpu.make_async_copy(
                    uQ.at[pb], out_hbm.at[cid, pl.ds(row0, _UCH)],
                    qsem.at[pb]).wait()
            else:
                pltpu.make_async_copy(
                    uQ.at[pb], q_sh.at[pl.ds(row0, _UCH)], qsem.at[pb]).wait()
                pltpu.make_async_copy(
                    uQ.at[pb], s_sh.at[pl.ds(row0, _UCH)], psem.at[pb]).wait()

        fire_loads(0, 0)

        def upd(c, carry):
            pb = lax.rem(c, 2)
            npb = 1 - pb
            r = row0 + c * _UCH

            @pl.when(c < _UPC - 1)
            def _():
                fire_loads(c + 1, npb)

            pltpu.make_async_copy(
                s_sh.at[pl.ds(row0, _UCH)], uS.at[pb], lsemS.at[pb]).wait()
            pltpu.make_async_copy(
                a_src.at[cid, pl.ds(row0, _UCH)], uA.at[pb],
                lsemA.at[pb]).wait()
            pltpu.make_async_copy(
                b_src.at[cid, pl.ds(row0, _UCH)], uB.at[pb],
                lsemB.at[pb]).wait()

            @pl.when(c >= 2)
            def _():
                wait_stores(pb)

            def rows(v, cc):
                for c3 in range(_CH // 16):
                    sl = pl.ds(c3 * 16, 16)
                    uQ[pb, v, sl] = uA[pb, v, sl] * uS[pb, v, sl] + uB[pb, v, sl]
                return cc

            lax.fori_loop(0, _UCH, rows, 0)
            if to_hbm:
                pltpu.async_copy(uQ.at[pb], out_hbm.at[cid, pl.ds(r, _UCH)],
                                 qsem.at[pb])
            else:
                pltpu.async_copy(uQ.at[pb], q_sh.at[pl.ds(r, _UCH)],
                                 qsem.at[pb])
                pltpu.async_copy(uQ.at[pb], s_sh.at[pl.ds(r, _UCH)],
                                 psem.at[pb])
            return carry

        lax.fori_loop(0, _UPC, upd, 0)
        for cc in (_UPC - 2, _UPC - 1):
            wait_stores(cc % 2)
        plsc.subcore_barrier()

    def step(k, carry):
        edge_pass()
        update(a_hbm, b_hbm, to_hbm=False)
        return carry

    lax.fori_loop(0, _K - 1, step, 0)
    edge_pass()
    update(a2_hbm, b2_hbm, to_hbm=True)


@jax.jit
def kernel(x, edge_index, W1, b1, W2, b2):
    src = edge_index[0]
    dst = edge_index[1]
    npad_ids = (jnp.arange(_EPAD - _E, dtype=jnp.int32) % (_NPAD - _N)) + _N
    src_p = jnp.concatenate([src, npad_ids])
    dst_p = jnp.concatenate([dst, npad_ids])
    dst_r = dst_p.reshape(_NT, _NCH, _CHUNK)
    pidx_r = (src_p * 16384 + dst_p).reshape(_NT, _NCH, _CHUNK)

    deg = _deg_kernel(dst_r)

    x_p = jnp.concatenate(
        [x, jnp.zeros((_NPAD - _N, _F), jnp.float32)], axis=0)
    A, B, A2, B2, Q0 = _prep_call(
        x_p, W1, b1.reshape(1, _HID), W2, b2.reshape(1, _C),
        deg.reshape(_NPAD, 1))

    out2 = _prop_kernel(pidx_r, A, B, A2, B2, Q0)
    return jnp.concatenate([out2[0], out2[1]], axis=1)[:_N, :_C]
